# baseline (speedup 1.0000x reference)
import jax
import jax.numpy as jnp
from jax.experimental import pallas as pl
from jax.experimental.pallas import tpu as pltpu

_B, _S, _D, _F, _E, _K = 32, 60, 1024, 4096, 8, 2
_SP = 64
_FT = 1024
_NF = _F // _FT
_P = _B * _K
_PP = _P + _E
_NC = _PP // 2
_M = 2 * _SP


def _router_body(xf_ref, ws_ref, bs_ref, pe_ref, ps_ref, pv_ref):
    logits = jnp.dot(xf_ref[...], ws_ref[...], preferred_element_type=jnp.float32)
    logits = logits + bs_ref[...]
    col = jax.lax.broadcasted_iota(jnp.int32, (_B, _E), 1)
    a1 = jnp.argmax(logits, axis=1).astype(jnp.int32)
    masked = jnp.where(col == a1[:, None], -jnp.inf, logits)
    a2 = jnp.argmax(masked, axis=1).astype(jnp.int32)
    m = (col == a1[:, None]) | (col == a2[:, None])
    mf = m.astype(jnp.float32)

    counts = jnp.sum(mf, axis=0, keepdims=True)
    odd = counts - 2.0 * jnp.floor(counts * 0.5)
    pad_counts = counts + odd
    ei = jax.lax.broadcasted_iota(jnp.int32, (_E, _E), 0)
    ej = jax.lax.broadcasted_iota(jnp.int32, (_E, _E), 1)
    upper = (ei < ej).astype(jnp.float32)
    off_pad = jnp.dot(pad_counts, upper,
                      preferred_element_type=jnp.float32)
    bi = jax.lax.broadcasted_iota(jnp.int32, (_B, _B), 0)
    bj = jax.lax.broadcasted_iota(jnp.int32, (_B, _B), 1)
    lower = (bj < bi).astype(jnp.float32)
    rank = jnp.dot(lower, mf, preferred_element_type=jnp.float32)
    pos = (off_pad + rank).astype(jnp.int32)

    slot = jax.lax.broadcasted_iota(jnp.int32, (_PP, _B, _E), 0)
    hit = jnp.where((slot == pos[None, :, :]) & m[None, :, :], 1.0, 0.0)
    brow = jax.lax.broadcasted_iota(jnp.int32, (_PP, _B, _E), 1).astype(jnp.float32)
    ecol = jax.lax.broadcasted_iota(jnp.int32, (_PP, _B, _E), 2).astype(jnp.float32)
    ps_out = jnp.sum(jnp.sum(hit * brow, axis=2), axis=1)
    pe_out = jnp.sum(jnp.sum(hit * ecol, axis=2), axis=1)
    pv_out = jnp.sum(jnp.sum(hit, axis=2), axis=1)

    slot2 = jax.lax.broadcasted_iota(jnp.int32, (_PP, _E), 0)
    erow = jax.lax.broadcasted_iota(jnp.int32, (_PP, _E), 1).astype(jnp.float32)
    padpos = (off_pad + counts).astype(jnp.int32)[0][None, :]
    hit2 = jnp.where((slot2 == padpos) & (odd[0][None, :] > 0.0), 1.0, 0.0)
    pe_out = pe_out + jnp.sum(hit2 * erow, axis=1)

    e_iota = jax.lax.broadcasted_iota(jnp.int32, (1, _E), 1).astype(jnp.float32)
    emax = jnp.max(jnp.where(counts > 0.0, e_iota, 0.0))
    total = jnp.sum(pad_counts).astype(jnp.int32)
    trailing = (slot2[:, 0] >= total).astype(jnp.float32)
    pe_out = pe_out + trailing * emax

    pe_ref[...] = pe_out.astype(jnp.int32)[None, :]
    ps_ref[...] = ps_out.astype(jnp.int32)[None, :]
    pv_ref[...] = pv_out.astype(jnp.int32)[None, :]


def _dispatch_body(ps_ref, xin_ref, xout_ref):
    xout_ref[...] = xin_ref[...]


def _moe_body(pe_ref, ps_ref, pv_ref, xd_ref, x_ref, w1_ref, b1_ref, w2_ref,
              b2_ref, g_ref, bt_ref, out_ref, acc_ref, h2_ref):
    f = pl.program_id(0)
    c = pl.program_id(1)
    cc = jnp.minimum(c, _NC - 1)
    cd = jnp.maximum(c - 2, 0)
    e_d = pe_ref[0, 2 * cd]
    b0d = ps_ref[0, 2 * cd]
    b1d = ps_ref[0, 2 * cd + 1]
    v0d = pv_ref[0, 2 * cd]
    v1d = pv_ref[0, 2 * cd + 1]
    v0c = pv_ref[0, 2 * cc]
    e_c = pe_ref[0, 2 * cc]

    @pl.when((f == 0) & (c == 0))
    def _init():
        acc_ref[...] = jnp.zeros_like(acc_ref)

    @pl.when((c >= 2) & (v0d > 0))
    def _drain():
        contrib = jnp.dot(h2_ref[c % 2], w2_ref[0],
                          preferred_element_type=jnp.float32)
        acc_ref[b0d] = acc_ref[b0d] + contrib[:_SP]

        @pl.when(v1d > 0)
        def _second():
            acc_ref[b1d] = acc_ref[b1d] + contrib[_SP:]

        @pl.when(f == 0)
        def _bias2():
            acc_ref[b0d] = acc_ref[b0d] + b2_ref[e_d][None, :]

            @pl.when(v1d > 0)
            def _bias2b():
                acc_ref[b1d] = acc_ref[b1d] + b2_ref[e_d][None, :]

    @pl.when((c < _NC) & (v0c > 0))
    def _compute():
        h = jnp.dot(xd_ref[0], w1_ref[0], preferred_element_type=jnp.float32)
        h = h + b1_ref[e_c, pl.ds(f * _FT, _FT)][None, :]
        h2_ref[c % 2] = 0.5 * h * (1.0 + jax.lax.erf(h * 0.7071067811865476))

    @pl.when((f == _NF - 1) & (c == _NC + 1))
    def _finish():
        z = x_ref[...] + acc_ref[...]
        mean = jnp.mean(z, axis=-1, keepdims=True)
        zc = z - mean
        var = jnp.mean(zc * zc, axis=-1, keepdims=True)
        res = zc * jax.lax.rsqrt(var + 1e-5) * g_ref[0] + bt_ref[0]
        out_ref[...] = res[:, :_S, :]


def kernel(x, W_switch, b_switch, W1, b1, W2, b2, gamma, beta):
    x_flat = x.reshape(_B, _S * _D)
    pe_pad, ps_pad, pv_pad = pl.pallas_call(
        _router_body,
        out_shape=(
            jax.ShapeDtypeStruct((1, _PP), jnp.int32),
            jax.ShapeDtypeStruct((1, _PP), jnp.int32),
            jax.ShapeDtypeStruct((1, _PP), jnp.int32),
        ),
    )(x_flat, W_switch, b_switch.reshape(1, _E))

    x_p = jnp.pad(x, ((0, 0), (0, _SP - _S), (0, 0)))

    disp_spec = pltpu.PrefetchScalarGridSpec(
        num_scalar_prefetch=1,
        grid=(_PP,),
        in_specs=[pl.BlockSpec((1, _SP, _D), lambda p, ps: (ps[0, p], 0, 0))],
        out_specs=pl.BlockSpec((1, _SP, _D), lambda p, ps: (p, 0, 0)),
    )
    x_disp = pl.pallas_call(
        _dispatch_body,
        grid_spec=disp_spec,
        out_shape=jax.ShapeDtypeStruct((_PP, _SP, _D), jnp.float32),
        compiler_params=pltpu.CompilerParams(
            dimension_semantics=("arbitrary",)),
    )(ps_pad, x_p)
    x_disp = x_disp.reshape(_NC, _M, _D)

    grid_spec = pltpu.PrefetchScalarGridSpec(
        num_scalar_prefetch=3,
        grid=(_NF, _NC + 2),
        in_specs=[
            pl.BlockSpec((1, _M, _D),
                         lambda f, c, pe, ps, pv: (jnp.minimum(c, _NC - 1), 0, 0)),
            pl.BlockSpec((_B, _SP, _D), lambda f, c, pe, ps, pv: (0, 0, 0)),
            pl.BlockSpec((1, _D, _FT),
                         lambda f, c, pe, ps, pv:
                         (pe[0, 2 * jnp.minimum(c, _NC - 1)], 0, f)),
            pl.BlockSpec((_E, _F), lambda f, c, pe, ps, pv: (0, 0)),
            pl.BlockSpec((1, _FT, _D),
                         lambda f, c, pe, ps, pv:
                         (pe[0, 2 * jnp.maximum(c - 2, 0)], f, 0)),
            pl.BlockSpec((_E, _D), lambda f, c, pe, ps, pv: (0, 0)),
            pl.BlockSpec((1, _D), lambda f, c, pe, ps, pv: (0, 0)),
            pl.BlockSpec((1, _D), lambda f, c, pe, ps, pv: (0, 0)),
        ],
        out_specs=pl.BlockSpec((_B, _S, _D), lambda f, c, pe, ps, pv: (0, 0, 0)),
        scratch_shapes=[
            pltpu.VMEM((_B, _SP, _D), jnp.float32),
            pltpu.VMEM((2, _M, _FT), jnp.float32),
        ],
    )
    out = pl.pallas_call(
        _moe_body,
        grid_spec=grid_spec,
        out_shape=jax.ShapeDtypeStruct((_B, _S, _D), jnp.float32),
        compiler_params=pltpu.CompilerParams(
            dimension_semantics=("arbitrary", "arbitrary")),
    )(pe_pad, ps_pad, pv_pad, x_disp, x_p, W1, b1, W2, b2,
      gamma.reshape(1, _D), beta.reshape(1, _D))
    return out


# speedup vs baseline: 1.0366x; 1.0366x over previous
import jax
import jax.numpy as jnp
from jax.experimental import pallas as pl
from jax.experimental.pallas import tpu as pltpu

_B, _S, _D, _F, _E, _K = 32, 60, 1024, 4096, 8, 2
_SP = 64
_FT = 1024
_NF = _F // _FT
_P = _B * _K
_PP = _P + _E
_NC = _PP // 2
_M = 2 * _SP


def _router_body(xf_ref, ws_ref, bs_ref, pe_ref, ps_ref, pv_ref):
    logits = jnp.dot(xf_ref[...], ws_ref[...], preferred_element_type=jnp.float32)
    logits = logits + bs_ref[...]
    col = jax.lax.broadcasted_iota(jnp.int32, (_B, _E), 1)
    a1 = jnp.argmax(logits, axis=1).astype(jnp.int32)
    masked = jnp.where(col == a1[:, None], -jnp.inf, logits)
    a2 = jnp.argmax(masked, axis=1).astype(jnp.int32)
    m = (col == a1[:, None]) | (col == a2[:, None])
    mf = m.astype(jnp.float32)

    counts = jnp.sum(mf, axis=0, keepdims=True)
    odd = counts - 2.0 * jnp.floor(counts * 0.5)
    pad_counts = counts + odd
    ei = jax.lax.broadcasted_iota(jnp.int32, (_E, _E), 0)
    ej = jax.lax.broadcasted_iota(jnp.int32, (_E, _E), 1)
    upper = (ei < ej).astype(jnp.float32)
    off_pad = jnp.dot(pad_counts, upper,
                      preferred_element_type=jnp.float32)
    bi = jax.lax.broadcasted_iota(jnp.int32, (_B, _B), 0)
    bj = jax.lax.broadcasted_iota(jnp.int32, (_B, _B), 1)
    lower = (bj < bi).astype(jnp.float32)
    rank = jnp.dot(lower, mf, preferred_element_type=jnp.float32)
    pos = (off_pad + rank).astype(jnp.int32)

    slot = jax.lax.broadcasted_iota(jnp.int32, (_PP, _B, _E), 0)
    hit = jnp.where((slot == pos[None, :, :]) & m[None, :, :], 1.0, 0.0)
    brow = jax.lax.broadcasted_iota(jnp.int32, (_PP, _B, _E), 1).astype(jnp.float32)
    ecol = jax.lax.broadcasted_iota(jnp.int32, (_PP, _B, _E), 2).astype(jnp.float32)
    ps_out = jnp.sum(jnp.sum(hit * brow, axis=2), axis=1)
    pe_out = jnp.sum(jnp.sum(hit * ecol, axis=2), axis=1)
    pv_out = jnp.sum(jnp.sum(hit, axis=2), axis=1)

    slot2 = jax.lax.broadcasted_iota(jnp.int32, (_PP, _E), 0)
    erow = jax.lax.broadcasted_iota(jnp.int32, (_PP, _E), 1).astype(jnp.float32)
    padpos = (off_pad + counts).astype(jnp.int32)[0][None, :]
    hit2 = jnp.where((slot2 == padpos) & (odd[0][None, :] > 0.0), 1.0, 0.0)
    pe_out = pe_out + jnp.sum(hit2 * erow, axis=1)

    e_iota = jax.lax.broadcasted_iota(jnp.int32, (1, _E), 1).astype(jnp.float32)
    emax = jnp.max(jnp.where(counts > 0.0, e_iota, 0.0))
    total = jnp.sum(pad_counts).astype(jnp.int32)
    trailing = (slot2[:, 0] >= total).astype(jnp.float32)
    pe_out = pe_out + trailing * emax

    pe_ref[...] = pe_out.astype(jnp.int32)[None, :]
    ps_ref[...] = ps_out.astype(jnp.int32)[None, :]
    pv_ref[...] = pv_out.astype(jnp.int32)[None, :]


def _dispatch_body(ps_ref, xin_ref, xout_ref):
    xout_ref[...] = xin_ref[...]


def _moe_body(pe_ref, ps_ref, pv_ref, xd_ref, x_ref, w1_ref, b1_ref, w2_ref,
              b2_ref, g_ref, bt_ref, out_ref, acc_ref):
    f = pl.program_id(0)
    c = pl.program_id(1)
    p0 = 2 * c
    e = pe_ref[0, p0]
    b0 = ps_ref[0, p0]
    b1v = ps_ref[0, p0 + 1]
    v0 = pv_ref[0, p0]
    v1 = pv_ref[0, p0 + 1]

    @pl.when((f == 0) & (c == 0))
    def _init():
        acc_ref[...] = jnp.zeros_like(acc_ref)

    @pl.when(v0 > 0)
    def _compute():
        h = jnp.dot(xd_ref[0], w1_ref[0], preferred_element_type=jnp.float32)
        h = h + b1_ref[e, pl.ds(f * _FT, _FT)][None, :]
        h = 0.5 * h * (1.0 + jax.lax.erf(h * 0.7071067811865476))
        contrib = jnp.dot(h, w2_ref[0], preferred_element_type=jnp.float32)
        acc_ref[b0] = acc_ref[b0] + contrib[:_SP]

        @pl.when(v1 > 0)
        def _second():
            acc_ref[b1v] = acc_ref[b1v] + contrib[_SP:]

        @pl.when(f == 0)
        def _bias2():
            acc_ref[b0] = acc_ref[b0] + b2_ref[e][None, :]

            @pl.when(v1 > 0)
            def _bias2b():
                acc_ref[b1v] = acc_ref[b1v] + b2_ref[e][None, :]

    @pl.when((f == _NF - 1) & (c == _NC - 1))
    def _finish():
        z = x_ref[...] + acc_ref[...]
        mean = jnp.mean(z, axis=-1, keepdims=True)
        zc = z - mean
        var = jnp.mean(zc * zc, axis=-1, keepdims=True)
        res = zc * jax.lax.rsqrt(var + 1e-5) * g_ref[0] + bt_ref[0]
        out_ref[...] = res[:, :_S, :]


def kernel(x, W_switch, b_switch, W1, b1, W2, b2, gamma, beta):
    x_flat = x.reshape(_B, _S * _D)
    pe_pad, ps_pad, pv_pad = pl.pallas_call(
        _router_body,
        out_shape=(
            jax.ShapeDtypeStruct((1, _PP), jnp.int32),
            jax.ShapeDtypeStruct((1, _PP), jnp.int32),
            jax.ShapeDtypeStruct((1, _PP), jnp.int32),
        ),
    )(x_flat, W_switch, b_switch.reshape(1, _E))

    x_p = jnp.pad(x, ((0, 0), (0, _SP - _S), (0, 0)))

    disp_spec = pltpu.PrefetchScalarGridSpec(
        num_scalar_prefetch=1,
        grid=(_PP,),
        in_specs=[pl.BlockSpec((1, _SP, _D), lambda p, ps: (ps[0, p], 0, 0))],
        out_specs=pl.BlockSpec((1, _SP, _D), lambda p, ps: (p, 0, 0)),
    )
    x_disp = pl.pallas_call(
        _dispatch_body,
        grid_spec=disp_spec,
        out_shape=jax.ShapeDtypeStruct((_PP, _SP, _D), jnp.float32),
        compiler_params=pltpu.CompilerParams(
            dimension_semantics=("arbitrary",)),
    )(ps_pad, x_p)
    x_disp = x_disp.reshape(_NC, _M, _D)

    grid_spec = pltpu.PrefetchScalarGridSpec(
        num_scalar_prefetch=3,
        grid=(_NF, _NC),
        in_specs=[
            pl.BlockSpec((1, _M, _D), lambda f, c, pe, ps, pv: (c, 0, 0)),
            pl.BlockSpec((_B, _SP, _D), lambda f, c, pe, ps, pv: (0, 0, 0)),
            pl.BlockSpec((1, _D, _FT),
                         lambda f, c, pe, ps, pv: (pe[0, 2 * c], 0, f)),
            pl.BlockSpec((_E, _F), lambda f, c, pe, ps, pv: (0, 0)),
            pl.BlockSpec((1, _FT, _D),
                         lambda f, c, pe, ps, pv: (pe[0, 2 * c], f, 0)),
            pl.BlockSpec((_E, _D), lambda f, c, pe, ps, pv: (0, 0)),
            pl.BlockSpec((1, _D), lambda f, c, pe, ps, pv: (0, 0)),
            pl.BlockSpec((1, _D), lambda f, c, pe, ps, pv: (0, 0)),
        ],
        out_specs=pl.BlockSpec((_B, _S, _D), lambda f, c, pe, ps, pv: (0, 0, 0)),
        scratch_shapes=[pltpu.VMEM((_B, _SP, _D), jnp.float32)],
    )
    out = pl.pallas_call(
        _moe_body,
        grid_spec=grid_spec,
        out_shape=jax.ShapeDtypeStruct((_B, _S, _D), jnp.float32),
        compiler_params=pltpu.CompilerParams(
            dimension_semantics=("arbitrary", "arbitrary")),
    )(pe_pad, ps_pad, pv_pad, x_disp, x_p, W1, b1, W2, b2,
      gamma.reshape(1, _D), beta.reshape(1, _D))
    return out


# speedup vs baseline: 1.2178x; 1.1748x over previous
import jax
import jax.numpy as jnp
from jax.experimental import pallas as pl
from jax.experimental.pallas import tpu as pltpu

_B, _S, _D, _F, _E, _K = 32, 60, 1024, 4096, 8, 2
_SP = 64
_FT = 1024
_NF = _F // _FT
_P = _B * _K
_PP = _P + _E
_NC = _PP // 2


def _router_body(xf_ref, ws_ref, bs_ref, pe_ref, ps_ref, pv_ref):
    logits = jnp.dot(xf_ref[...], ws_ref[...], preferred_element_type=jnp.float32)
    logits = logits + bs_ref[...]
    col = jax.lax.broadcasted_iota(jnp.int32, (_B, _E), 1)
    a1 = jnp.argmax(logits, axis=1).astype(jnp.int32)
    masked = jnp.where(col == a1[:, None], -jnp.inf, logits)
    a2 = jnp.argmax(masked, axis=1).astype(jnp.int32)
    m = (col == a1[:, None]) | (col == a2[:, None])
    mf = m.astype(jnp.float32)

    counts = jnp.sum(mf, axis=0, keepdims=True)
    odd = counts - 2.0 * jnp.floor(counts * 0.5)
    pad_counts = counts + odd
    ei = jax.lax.broadcasted_iota(jnp.int32, (_E, _E), 0)
    ej = jax.lax.broadcasted_iota(jnp.int32, (_E, _E), 1)
    upper = (ei < ej).astype(jnp.float32)
    off_pad = jnp.dot(pad_counts, upper,
                      preferred_element_type=jnp.float32)
    bi = jax.lax.broadcasted_iota(jnp.int32, (_B, _B), 0)
    bj = jax.lax.broadcasted_iota(jnp.int32, (_B, _B), 1)
    lower = (bj < bi).astype(jnp.float32)
    rank = jnp.dot(lower, mf, preferred_element_type=jnp.float32)
    pos = (off_pad + rank).astype(jnp.int32)

    slot = jax.lax.broadcasted_iota(jnp.int32, (_PP, _B, _E), 0)
    hit = jnp.where((slot == pos[None, :, :]) & m[None, :, :], 1.0, 0.0)
    brow = jax.lax.broadcasted_iota(jnp.int32, (_PP, _B, _E), 1).astype(jnp.float32)
    ecol = jax.lax.broadcasted_iota(jnp.int32, (_PP, _B, _E), 2).astype(jnp.float32)
    ps_out = jnp.sum(jnp.sum(hit * brow, axis=2), axis=1)
    pe_out = jnp.sum(jnp.sum(hit * ecol, axis=2), axis=1)
    pv_out = jnp.sum(jnp.sum(hit, axis=2), axis=1)

    slot2 = jax.lax.broadcasted_iota(jnp.int32, (_PP, _E), 0)
    erow = jax.lax.broadcasted_iota(jnp.int32, (_PP, _E), 1).astype(jnp.float32)
    padpos = (off_pad + counts).astype(jnp.int32)[0][None, :]
    hit2 = jnp.where((slot2 == padpos) & (odd[0][None, :] > 0.0), 1.0, 0.0)
    pe_out = pe_out + jnp.sum(hit2 * erow, axis=1)

    e_iota = jax.lax.broadcasted_iota(jnp.int32, (1, _E), 1).astype(jnp.float32)
    emax = jnp.max(jnp.where(counts > 0.0, e_iota, 0.0))
    total = jnp.sum(pad_counts).astype(jnp.int32)
    trailing = (slot2[:, 0] >= total).astype(jnp.float32)
    pe_out = pe_out + trailing * emax

    pe_ref[...] = pe_out.astype(jnp.int32)[None, :]
    ps_ref[...] = ps_out.astype(jnp.int32)[None, :]
    pv_ref[...] = pv_out.astype(jnp.int32)[None, :]


def _moe_body(pe_ref, ps_ref, pv_ref, x_ref, w1_ref, b1_ref, w2_ref,
              b2_ref, g_ref, bt_ref, out_ref, acc_ref):
    f = pl.program_id(0)
    c = pl.program_id(1)
    p0 = 2 * c
    e = pe_ref[0, p0]
    b0 = ps_ref[0, p0]
    b1v = ps_ref[0, p0 + 1]
    v0 = pv_ref[0, p0]
    v1 = pv_ref[0, p0 + 1]

    @pl.when((f == 0) & (c == 0))
    def _init():
        acc_ref[...] = jnp.zeros_like(acc_ref)

    @pl.when(v0 > 0)
    def _compute():
        xb = jnp.concatenate([x_ref[b0], x_ref[b1v]], axis=0)
        h = jnp.dot(xb, w1_ref[0], preferred_element_type=jnp.float32)
        h = h + b1_ref[e, pl.ds(f * _FT, _FT)][None, :]
        h = 0.5 * h * (1.0 + jax.lax.erf(h * 0.7071067811865476))
        contrib = jnp.dot(h, w2_ref[0], preferred_element_type=jnp.float32)
        acc_ref[b0] = acc_ref[b0] + contrib[:_SP]

        @pl.when(v1 > 0)
        def _second():
            acc_ref[b1v] = acc_ref[b1v] + contrib[_SP:]

        @pl.when(f == 0)
        def _bias2():
            acc_ref[b0] = acc_ref[b0] + b2_ref[e][None, :]

            @pl.when(v1 > 0)
            def _bias2b():
                acc_ref[b1v] = acc_ref[b1v] + b2_ref[e][None, :]

    @pl.when((f == _NF - 1) & (c == _NC - 1))
    def _finish():
        z = x_ref[...] + acc_ref[...]
        mean = jnp.mean(z, axis=-1, keepdims=True)
        zc = z - mean
        var = jnp.mean(zc * zc, axis=-1, keepdims=True)
        res = zc * jax.lax.rsqrt(var + 1e-5) * g_ref[0] + bt_ref[0]
        out_ref[...] = res[:, :_S, :]


def kernel(x, W_switch, b_switch, W1, b1, W2, b2, gamma, beta):
    x_flat = x.reshape(_B, _S * _D)
    pe_pad, ps_pad, pv_pad = pl.pallas_call(
        _router_body,
        out_shape=(
            jax.ShapeDtypeStruct((1, _PP), jnp.int32),
            jax.ShapeDtypeStruct((1, _PP), jnp.int32),
            jax.ShapeDtypeStruct((1, _PP), jnp.int32),
        ),
    )(x_flat, W_switch, b_switch.reshape(1, _E))

    x_p = jnp.pad(x, ((0, 0), (0, _SP - _S), (0, 0)))

    grid_spec = pltpu.PrefetchScalarGridSpec(
        num_scalar_prefetch=3,
        grid=(_NF, _NC),
        in_specs=[
            pl.BlockSpec((_B, _SP, _D), lambda f, c, pe, ps, pv: (0, 0, 0)),
            pl.BlockSpec((1, _D, _FT), lambda f, c, pe, ps, pv: (pe[0, 2 * c], 0, f)),
            pl.BlockSpec((_E, _F), lambda f, c, pe, ps, pv: (0, 0)),
            pl.BlockSpec((1, _FT, _D), lambda f, c, pe, ps, pv: (pe[0, 2 * c], f, 0)),
            pl.BlockSpec((_E, _D), lambda f, c, pe, ps, pv: (0, 0)),
            pl.BlockSpec((1, _D), lambda f, c, pe, ps, pv: (0, 0)),
            pl.BlockSpec((1, _D), lambda f, c, pe, ps, pv: (0, 0)),
        ],
        out_specs=pl.BlockSpec((_B, _S, _D), lambda f, c, pe, ps, pv: (0, 0, 0)),
        scratch_shapes=[pltpu.VMEM((_B, _SP, _D), jnp.float32)],
    )
    out = pl.pallas_call(
        _moe_body,
        grid_spec=grid_spec,
        out_shape=jax.ShapeDtypeStruct((_B, _S, _D), jnp.float32),
        compiler_params=pltpu.CompilerParams(
            dimension_semantics=("arbitrary", "arbitrary")),
    )(pe_pad, ps_pad, pv_pad, x_p, W1, b1, W2, b2,
      gamma.reshape(1, _D), beta.reshape(1, _D))
    return out


# speedup vs baseline: 1.4148x; 1.1618x over previous
import jax
import jax.numpy as jnp
from jax.experimental import pallas as pl
from jax.experimental.pallas import tpu as pltpu

_B, _S, _D, _F, _E, _K = 32, 60, 1024, 4096, 8, 2
_SP = 64
_FT = 2048
_NF = _F // _FT
_P = _B * _K
_PP = _P + _E
_NC = _PP // 2


def _router_body(xf_ref, ws_ref, bs_ref, pe_ref, ps_ref, pv_ref):
    logits = jnp.dot(xf_ref[...], ws_ref[...], preferred_element_type=jnp.float32)
    logits = logits + bs_ref[...]
    col = jax.lax.broadcasted_iota(jnp.int32, (_B, _E), 1)
    a1 = jnp.argmax(logits, axis=1).astype(jnp.int32)
    masked = jnp.where(col == a1[:, None], -jnp.inf, logits)
    a2 = jnp.argmax(masked, axis=1).astype(jnp.int32)
    m = (col == a1[:, None]) | (col == a2[:, None])
    mf = m.astype(jnp.float32)

    counts = jnp.sum(mf, axis=0, keepdims=True)
    odd = counts - 2.0 * jnp.floor(counts * 0.5)
    pad_counts = counts + odd
    ei = jax.lax.broadcasted_iota(jnp.int32, (_E, _E), 0)
    ej = jax.lax.broadcasted_iota(jnp.int32, (_E, _E), 1)
    upper = (ei < ej).astype(jnp.float32)
    off_pad = jnp.dot(pad_counts, upper,
                      preferred_element_type=jnp.float32)
    bi = jax.lax.broadcasted_iota(jnp.int32, (_B, _B), 0)
    bj = jax.lax.broadcasted_iota(jnp.int32, (_B, _B), 1)
    lower = (bj < bi).astype(jnp.float32)
    rank = jnp.dot(lower, mf, preferred_element_type=jnp.float32)
    pos = (off_pad + rank).astype(jnp.int32)

    slot = jax.lax.broadcasted_iota(jnp.int32, (_PP, _B, _E), 0)
    hit = jnp.where((slot == pos[None, :, :]) & m[None, :, :], 1.0, 0.0)
    brow = jax.lax.broadcasted_iota(jnp.int32, (_PP, _B, _E), 1).astype(jnp.float32)
    ecol = jax.lax.broadcasted_iota(jnp.int32, (_PP, _B, _E), 2).astype(jnp.float32)
    ps_out = jnp.sum(jnp.sum(hit * brow, axis=2), axis=1)
    pe_out = jnp.sum(jnp.sum(hit * ecol, axis=2), axis=1)
    pv_out = jnp.sum(jnp.sum(hit, axis=2), axis=1)

    slot2 = jax.lax.broadcasted_iota(jnp.int32, (_PP, _E), 0)
    erow = jax.lax.broadcasted_iota(jnp.int32, (_PP, _E), 1).astype(jnp.float32)
    padpos = (off_pad + counts).astype(jnp.int32)[0][None, :]
    hit2 = jnp.where((slot2 == padpos) & (odd[0][None, :] > 0.0), 1.0, 0.0)
    pe_out = pe_out + jnp.sum(hit2 * erow, axis=1)

    e_iota = jax.lax.broadcasted_iota(jnp.int32, (1, _E), 1).astype(jnp.float32)
    emax = jnp.max(jnp.where(counts > 0.0, e_iota, 0.0))
    total = jnp.sum(pad_counts).astype(jnp.int32)
    trailing = (slot2[:, 0] >= total).astype(jnp.float32)
    pe_out = pe_out + trailing * emax

    pe_ref[...] = pe_out.astype(jnp.int32)[None, :]
    ps_ref[...] = ps_out.astype(jnp.int32)[None, :]
    pv_ref[...] = pv_out.astype(jnp.int32)[None, :]


def _moe_body(pe_ref, ps_ref, pv_ref, x_ref, w1_ref, b1_ref, w2_ref,
              b2_ref, g_ref, bt_ref, out_ref):
    f = pl.program_id(0)
    c = pl.program_id(1)
    p0 = 2 * c
    e = pe_ref[0, p0]
    b0 = ps_ref[0, p0]
    b1v = ps_ref[0, p0 + 1]
    v0 = pv_ref[0, p0]
    v1 = pv_ref[0, p0 + 1]

    @pl.when((f == 0) & (c == 0))
    def _init():
        out_ref[...] = x_ref[:, :_S, :]

    @pl.when(v0 > 0)
    def _compute():
        xb = jnp.concatenate([x_ref[b0], x_ref[b1v]], axis=0)
        h = jnp.dot(xb, w1_ref[0], preferred_element_type=jnp.float32)
        h = h + b1_ref[e, pl.ds(f * _FT, _FT)][None, :]
        h = 0.5 * h * (1.0 + jax.lax.erf(h * 0.7071067811865476))
        contrib = jnp.dot(h, w2_ref[0], preferred_element_type=jnp.float32)
        out_ref[b0] = out_ref[b0] + contrib[:_S]

        @pl.when(v1 > 0)
        def _second():
            out_ref[b1v] = out_ref[b1v] + contrib[_SP:_SP + _S]

        @pl.when(f == 0)
        def _bias2():
            out_ref[b0] = out_ref[b0] + b2_ref[e][None, :]

            @pl.when(v1 > 0)
            def _bias2b():
                out_ref[b1v] = out_ref[b1v] + b2_ref[e][None, :]

    @pl.when((f == _NF - 1) & (c == _NC - 1))
    def _finish():
        z = out_ref[...]
        mean = jnp.mean(z, axis=-1, keepdims=True)
        zc = z - mean
        var = jnp.mean(zc * zc, axis=-1, keepdims=True)
        out_ref[...] = zc * jax.lax.rsqrt(var + 1e-5) * g_ref[0] + bt_ref[0]


def kernel(x, W_switch, b_switch, W1, b1, W2, b2, gamma, beta):
    x_flat = x.reshape(_B, _S * _D)
    pe_pad, ps_pad, pv_pad = pl.pallas_call(
        _router_body,
        out_shape=(
            jax.ShapeDtypeStruct((1, _PP), jnp.int32),
            jax.ShapeDtypeStruct((1, _PP), jnp.int32),
            jax.ShapeDtypeStruct((1, _PP), jnp.int32),
        ),
    )(x_flat, W_switch, b_switch.reshape(1, _E))

    x_p = jnp.pad(x, ((0, 0), (0, _SP - _S), (0, 0)))

    grid_spec = pltpu.PrefetchScalarGridSpec(
        num_scalar_prefetch=3,
        grid=(_NF, _NC),
        in_specs=[
            pl.BlockSpec((_B, _SP, _D), lambda f, c, pe, ps, pv: (0, 0, 0)),
            pl.BlockSpec((1, _D, _FT), lambda f, c, pe, ps, pv: (pe[0, 2 * c], 0, f)),
            pl.BlockSpec((_E, _F), lambda f, c, pe, ps, pv: (0, 0)),
            pl.BlockSpec((1, _FT, _D), lambda f, c, pe, ps, pv: (pe[0, 2 * c], f, 0)),
            pl.BlockSpec((_E, _D), lambda f, c, pe, ps, pv: (0, 0)),
            pl.BlockSpec((1, _D), lambda f, c, pe, ps, pv: (0, 0)),
            pl.BlockSpec((1, _D), lambda f, c, pe, ps, pv: (0, 0)),
        ],
        out_specs=pl.BlockSpec((_B, _S, _D), lambda f, c, pe, ps, pv: (0, 0, 0)),
    )
    out = pl.pallas_call(
        _moe_body,
        grid_spec=grid_spec,
        out_shape=jax.ShapeDtypeStruct((_B, _S, _D), jnp.float32),
        compiler_params=pltpu.CompilerParams(
            dimension_semantics=("arbitrary", "arbitrary")),
    )(pe_pad, ps_pad, pv_pad, x_p, W1, b1, W2, b2,
      gamma.reshape(1, _D), beta.reshape(1, _D))
    return out
